# baseline (speedup 1.0000x reference)
import functools

import jax
import jax.numpy as jnp
from jax import lax
from jax.experimental import pallas as pl
from jax.experimental.pallas import tpu as pltpu

_EPS = 1e-20
_KS1 = 42
_KS2 = 0x1BD11BDA ^ 42
_ROT_A = (13, 15, 26, 6)
_ROT_B = (17, 29, 16, 24)


def _rotl(x, d):
    return (x << jnp.uint32(d)) | (x >> jnp.uint32(32 - d))


def _rounds(x0, x1, rots):
    for d in rots:
        x0 = x0 + x1
        x1 = _rotl(x1, d)
        x1 = x1 ^ x0
    return x0, x1


def _threefry_bits(lo):
    ks1 = jnp.uint32(_KS1)
    ks2 = jnp.uint32(_KS2)
    x1 = lo + ks1
    x0 = jnp.zeros_like(lo)
    x0, x1 = _rounds(x0, x1, _ROT_A)
    x0 = x0 + ks1
    x1 = x1 + jnp.uint32(_KS2 + 1)
    x0, x1 = _rounds(x0, x1, _ROT_B)
    x0 = x0 + ks2
    x1 = x1 + jnp.uint32(2)
    x0, x1 = _rounds(x0, x1, _ROT_A)
    x1 = x1 + jnp.uint32(_KS1 + 3)
    x0, x1 = _rounds(x0, x1, _ROT_B)
    x0 = x0 + ks1
    x1 = x1 + jnp.uint32(_KS2 + 4)
    x0, x1 = _rounds(x0, x1, _ROT_A)
    x0 = x0 + ks2
    x1 = x1 + jnp.uint32(5)
    return x0 ^ x1


def _body(x_ref, y_ref, *, n_cols):
    shape = x_ref.shape
    _, s_dim, l_dim = shape
    row0 = pl.program_id(0) * shape[0]
    ri = lax.broadcasted_iota(jnp.int32, shape, 0)
    si = lax.broadcasted_iota(jnp.int32, shape, 1)
    li = lax.broadcasted_iota(jnp.int32, shape, 2)
    lin = (row0 + ri) * n_cols + si * l_dim + li
    bits = _threefry_bits(lin.astype(jnp.uint32))
    fbits = (bits >> jnp.uint32(9)) | jnp.uint32(0x3F800000)
    u = lax.bitcast_convert_type(fbits, jnp.float32) - jnp.float32(1.0)
    g = -jnp.log(-jnp.log(u + _EPS) + _EPS)
    z = x_ref[...] + g
    m = jnp.max(z, axis=(1, 2), keepdims=True)
    e = jnp.exp(z - m)
    denom = jnp.sum(e, axis=(1, 2), keepdims=True)
    y_ref[...] = e / denom


def kernel(x):
    b_dim, n_cols = x.shape
    s_dim = 8
    l_dim = n_cols // s_dim
    r_dim = 2
    xr = x.reshape(b_dim, s_dim, l_dim)
    y = pl.pallas_call(
        functools.partial(_body, n_cols=n_cols),
        grid=(b_dim // r_dim,),
        in_specs=[pl.BlockSpec((r_dim, s_dim, l_dim), lambda i: (i, 0, 0))],
        out_specs=pl.BlockSpec((r_dim, s_dim, l_dim), lambda i: (i, 0, 0)),
        out_shape=jax.ShapeDtypeStruct((b_dim, s_dim, l_dim), x.dtype),
    )(xr)
    return y.reshape(b_dim, n_cols)
